# Initial kernel scaffold; baseline (speedup 1.0000x reference)
import jax
import jax.numpy as jnp
from jax.experimental import pallas as pl


def kernel(x, emb_table, W1, b1, W2, b2):
    raise NotImplementedError("write your pallas kernel here")


# speedup vs baseline: 6.2275x; 6.2275x over previous
import functools

import jax
import jax.numpy as jnp
from jax import lax
from jax.experimental import pallas as pl
from jax.experimental.pallas import tpu as pltpu
from jax.experimental.pallas import tpu_sc as plsc

B = 4096
L = 50
E = 128
HIDDEN = 4096
CLASSES = 1024

NUM_CORES = 2
NUM_SUBCORES = 16
NW = NUM_CORES * NUM_SUBCORES
RPW = B // NW
CHUNK = 4
NCH = RPW // CHUNK
LANES = 16


def _pool_body(idx_hbm, table_hbm, out_hbm, idx_v, rows_v, out_v, sem):
    wid = lax.axis_index("s") * NUM_CORES + lax.axis_index("c")
    base = wid * (RPW * L)
    pltpu.sync_copy(idx_hbm.at[pl.ds(base, RPW * L)], idx_v)

    @pl.loop(0, NCH)
    def _(g):
        pltpu.async_copy(
            table_hbm.at[idx_v.at[pl.ds(g * (CHUNK * L), CHUNK * L)]],
            rows_v,
            sem,
        ).wait()
        for j in range(CHUNK):
            def body(r, accs):
                return tuple(
                    a + rows_v[j * L + r, pl.ds(c * LANES, LANES)]
                    for c, a in enumerate(accs)
                )
            accs = tuple(jnp.zeros((LANES,), jnp.float32) for _ in range(E // LANES))
            accs = lax.fori_loop(0, L, body, accs)
            for c in range(E // LANES):
                out_v[g * CHUNK + j, pl.ds(c * LANES, LANES)] = accs[c] * (1.0 / L)

    pltpu.sync_copy(out_v, out_hbm.at[pl.ds(wid * RPW, RPW)])


@jax.jit
def _pool(idx, table):
    k = pl.kernel(
        _pool_body,
        out_type=jax.ShapeDtypeStruct((B, E), jnp.float32),
        mesh=plsc.VectorSubcoreMesh(core_axis_name="c", subcore_axis_name="s"),
        scratch_types=[
            pltpu.VMEM((RPW * L,), jnp.int32),
            pltpu.VMEM((CHUNK * L, E), jnp.float32),
            pltpu.VMEM((RPW, E), jnp.float32),
            pltpu.SemaphoreType.DMA,
        ],
    )
    return k(idx, table)


BB = 512


def _mlp_body(p_ref, w1_ref, b1_ref, w2_ref, b2_ref, o_ref):
    h = jnp.dot(p_ref[...], w1_ref[...], preferred_element_type=jnp.float32)
    h = jnp.maximum(h + b1_ref[...], 0.0)
    logits = jnp.dot(h, w2_ref[...], preferred_element_type=jnp.float32)
    logits = logits + b2_ref[...]
    m = jnp.max(logits, axis=1, keepdims=True)
    s = logits - m
    lse = jnp.log(jnp.sum(jnp.exp(s), axis=1, keepdims=True))
    o_ref[...] = s - lse


@jax.jit
def _mlp(pooled, W1, b1, W2, b2):
    return pl.pallas_call(
        _mlp_body,
        grid=(B // BB,),
        in_specs=[
            pl.BlockSpec((BB, E), lambda i: (i, 0)),
            pl.BlockSpec((E, HIDDEN), lambda i: (0, 0)),
            pl.BlockSpec((1, HIDDEN), lambda i: (0, 0)),
            pl.BlockSpec((HIDDEN, CLASSES), lambda i: (0, 0)),
            pl.BlockSpec((1, CLASSES), lambda i: (0, 0)),
        ],
        out_specs=pl.BlockSpec((BB, CLASSES), lambda i: (i, 0)),
        out_shape=jax.ShapeDtypeStruct((B, CLASSES), jnp.float32),
    )(pooled, W1, b1, W2, b2)


def kernel(x, emb_table, W1, b1, W2, b2):
    idx = x.reshape(-1).astype(jnp.int32)
    pooled = _pool(idx, emb_table)
    return _mlp(pooled, W1, b1.reshape(1, HIDDEN), W2, b2.reshape(1, CLASSES))


# speedup vs baseline: 8.0940x; 1.2997x over previous
import functools

import jax
import jax.numpy as jnp
from jax import lax
from jax.experimental import pallas as pl
from jax.experimental.pallas import tpu as pltpu
from jax.experimental.pallas import tpu_sc as plsc

B = 4096
L = 50
E = 128
HIDDEN = 4096
CLASSES = 1024

NUM_CORES = 2
NUM_SUBCORES = 16
NW = NUM_CORES * NUM_SUBCORES
RPW = B // NW
CHUNK = 4
NCH = RPW // CHUNK
LANES = 16


def _pool_body(idx_hbm, table_hbm, out_hbm, idx_v, rows0_v, rows1_v, out_v,
               sem0, sem1):
    wid = lax.axis_index("s") * NUM_CORES + lax.axis_index("c")
    base = wid * (RPW * L)
    pltpu.sync_copy(idx_hbm.at[pl.ds(base, RPW * L)], idx_v)

    bufs = (rows0_v, rows1_v)
    sems = (sem0, sem1)

    def start(g, b):
        pltpu.make_async_copy(
            table_hbm.at[idx_v.at[pl.ds(g * (CHUNK * L), CHUNK * L)]],
            bufs[b], sems[b],
        ).start()

    def wait(b):
        pltpu.make_async_copy(
            table_hbm.at[idx_v.at[pl.ds(0, CHUNK * L)]],
            bufs[b], sems[b],
        ).wait()

    def reduce_chunk(g, b):
        rows_v = bufs[b]
        for j in range(CHUNK):
            def body(r, accs):
                return tuple(
                    a + rows_v[j * L + r, pl.ds(c * LANES, LANES)]
                    for c, a in enumerate(accs)
                )
            accs = tuple(jnp.zeros((LANES,), jnp.float32) for _ in range(E // LANES))
            accs = lax.fori_loop(0, L, body, accs, unroll=5)
            for c in range(E // LANES):
                out_v[g * CHUNK + j, pl.ds(c * LANES, LANES)] = accs[c] * (1.0 / L)

    start(0, 0)
    start(1, 1)

    @pl.loop(0, NCH, step=2)
    def _(g):
        for b in range(2):
            gg = g + b
            wait(b)
            reduce_chunk(gg, b)

            @pl.when(gg + 2 < NCH)
            def _():
                start(gg + 2, b)

    pltpu.sync_copy(out_v, out_hbm.at[pl.ds(wid * RPW, RPW)])


@jax.jit
def _pool(idx, table):
    k = pl.kernel(
        _pool_body,
        out_type=jax.ShapeDtypeStruct((B, E), jnp.float32),
        mesh=plsc.VectorSubcoreMesh(core_axis_name="c", subcore_axis_name="s"),
        scratch_types=[
            pltpu.VMEM((RPW * L,), jnp.int32),
            pltpu.VMEM((CHUNK * L, E), jnp.float32),
            pltpu.VMEM((CHUNK * L, E), jnp.float32),
            pltpu.VMEM((RPW, E), jnp.float32),
            pltpu.SemaphoreType.DMA,
            pltpu.SemaphoreType.DMA,
        ],
    )
    return k(idx, table)


BB = 512


def _mlp_body(p_ref, w1_ref, b1_ref, w2_ref, b2_ref, o_ref):
    h = jnp.dot(p_ref[...], w1_ref[...], preferred_element_type=jnp.float32)
    h = jnp.maximum(h + b1_ref[...], 0.0)
    logits = jnp.dot(h, w2_ref[...], preferred_element_type=jnp.float32)
    logits = logits + b2_ref[...]
    m = jnp.max(logits, axis=1, keepdims=True)
    s = logits - m
    lse = jnp.log(jnp.sum(jnp.exp(s), axis=1, keepdims=True))
    o_ref[...] = s - lse


@jax.jit
def _mlp(pooled, W1, b1, W2, b2):
    return pl.pallas_call(
        _mlp_body,
        grid=(B // BB,),
        in_specs=[
            pl.BlockSpec((BB, E), lambda i: (i, 0)),
            pl.BlockSpec((E, HIDDEN), lambda i: (0, 0)),
            pl.BlockSpec((1, HIDDEN), lambda i: (0, 0)),
            pl.BlockSpec((HIDDEN, CLASSES), lambda i: (0, 0)),
            pl.BlockSpec((1, CLASSES), lambda i: (0, 0)),
        ],
        out_specs=pl.BlockSpec((BB, CLASSES), lambda i: (i, 0)),
        out_shape=jax.ShapeDtypeStruct((B, CLASSES), jnp.float32),
    )(pooled, W1, b1, W2, b2)


def kernel(x, emb_table, W1, b1, W2, b2):
    idx = x.reshape(-1).astype(jnp.int32)
    pooled = _pool(idx, emb_table)
    return _mlp(pooled, W1, b1.reshape(1, HIDDEN), W2, b2.reshape(1, CLASSES))
